# baseline (speedup 1.0000x reference)
import functools

import jax
import jax.numpy as jnp
from jax import lax
from jax.experimental import pallas as pl
from jax.experimental.pallas import tpu as pltpu
from jax.experimental.pallas import tpu_sc as plsc

_REGION = (21, 6, 7, 8, 9, 10, 13, 14, 15, 16, 19, 20,
           22, 25, 26, 27, 28, 31, 32, 33, 34, 35)
_RUNS = (
    (21, 0, 1),
    (6, 1, 5),
    (13, 6, 4),
    (19, 10, 2),
    (22, 12, 1),
    (25, 13, 4),
    (31, 17, 5),
)
_C_USED = 40
_C_OUT = 22
_L = 16
_NSLOTS = 4
_WMAIN = 896
_B_SC = 224


def kernel(x):
    B, _, C_in, W = x.shape
    x3 = x.reshape(B, C_in, W)
    region = jnp.array(_REGION, dtype=jnp.int32)
    b_tc = B - _B_SC

    info = plsc.get_sparse_core_info()
    nc, ns = info.num_cores, info.num_subcores
    nw = nc * ns
    bpw = _B_SC // nw
    wtail = W - _WMAIN
    ntf = wtail // _L
    ttail = wtail - _L

    mesh = plsc.VectorSubcoreMesh(core_axis_name="c", subcore_axis_name="s")

    @functools.partial(
        pl.kernel,
        out_type=jax.ShapeDtypeStruct((B, _C_OUT, W), x.dtype),
        mesh=mesh,
        scratch_types=[
            pltpu.VMEM((_NSLOTS, _C_OUT, _WMAIN), jnp.float32),
            pltpu.VMEM((_NSLOTS, _C_USED, wtail), jnp.float32),
            pltpu.VMEM((_NSLOTS, _C_OUT, wtail), jnp.float32),
            pltpu.VMEM((_C_OUT,), jnp.int32),
            [pltpu.SemaphoreType.DMA] * _NSLOTS,
            [pltpu.SemaphoreType.DMA] * _NSLOTS,
            [pltpu.SemaphoreType.DMA] * _NSLOTS,
            [pltpu.SemaphoreType.DMA] * _NSLOTS,
        ],
    )
    def sc_gather(x_hbm, region_hbm, out_hbm, slab, tin, tout, idx,
                  g_sems, t_sems, om_sems, ot_sems):
        wid = lax.axis_index("s") * nc + lax.axis_index("c")
        b0 = b_tc + wid * bpw

        pltpu.sync_copy(region_hbm, idx)

        def g_desc(i):
            s = i % _NSLOTS
            return pltpu.make_async_copy(
                x_hbm.at[b0 + i, :, pl.ds(0, _WMAIN)].at[idx],
                slab.at[s], g_sems[s])

        def t_desc(i):
            s = i % _NSLOTS
            return pltpu.make_async_copy(
                x_hbm.at[b0 + i, pl.ds(0, _C_USED), pl.ds(_WMAIN, wtail)],
                tin.at[s], t_sems[s])

        def om_desc(i):
            s = i % _NSLOTS
            return pltpu.make_async_copy(
                slab.at[s], out_hbm.at[b0 + i, :, pl.ds(0, _WMAIN)],
                om_sems[s])

        def ot_desc(i):
            s = i % _NSLOTS
            return pltpu.make_async_copy(
                tout.at[s], out_hbm.at[b0 + i, :, pl.ds(_WMAIN, wtail)],
                ot_sems[s])

        def permute_tail(s):
            def chunk(k, carry):
                off = k * _L
                for j, r in enumerate(_REGION):
                    tout[s, j, pl.ds(off, _L)] = tin[s, r, pl.ds(off, _L)]
                return carry
            lax.fori_loop(0, ntf, chunk, 0)
            for j, r in enumerate(_REGION):
                tout[s, j, pl.ds(ttail, _L)] = tin[s, r, pl.ds(ttail, _L)]

        for i in range(_NSLOTS - 1):
            g_desc(i).start()
            t_desc(i).start()
        for i in range(bpw):
            g_desc(i).wait()
            om_desc(i).start()
            t_desc(i).wait()
            permute_tail(i % _NSLOTS)
            ot_desc(i).start()
            nxt = i + _NSLOTS - 1
            if nxt < bpw:
                if nxt >= _NSLOTS:
                    om_desc(nxt - _NSLOTS).wait()
                    ot_desc(nxt - _NSLOTS).wait()
                g_desc(nxt).start()
                t_desc(nxt).start()
        for i in range(max(bpw - _NSLOTS, 0), bpw):
            om_desc(i).wait()
            ot_desc(i).wait()

    def tc_body(x_ref, sc_ref, o_ref):
        del sc_ref
        for i0, o0, ln in _RUNS:
            o_ref[:, o0:o0 + ln, :] = x_ref[:, i0:i0 + ln, :]

    sc_out = sc_gather(x3, region)

    nb = 50
    out = pl.pallas_call(
        tc_body,
        grid=(b_tc // nb,),
        in_specs=[
            pl.BlockSpec((nb, C_in, W), lambda i: (i, 0, 0)),
            pl.BlockSpec(memory_space=pl.ANY),
        ],
        out_specs=pl.BlockSpec((nb, _C_OUT, W), lambda i: (i, 0, 0)),
        out_shape=jax.ShapeDtypeStruct((B, _C_OUT, W), x.dtype),
        input_output_aliases={1: 0},
    )(x3, sc_out)
    return out.reshape(B, 1, _C_OUT, W)


# speedup vs baseline: 1.0393x; 1.0393x over previous
import functools

import jax
import jax.numpy as jnp
from jax import lax
from jax.experimental import pallas as pl
from jax.experimental.pallas import tpu as pltpu
from jax.experimental.pallas import tpu_sc as plsc

_REGION = (21, 6, 7, 8, 9, 10, 13, 14, 15, 16, 19, 20,
           22, 25, 26, 27, 28, 31, 32, 33, 34, 35)
_C_USED = 40
_C_OUT = 22
_L = 16
_NSLOTS = 4
_WMAIN = 896


def kernel(x):
    B, _, C_in, W = x.shape
    x3 = x.reshape(B, C_in, W)
    region = jnp.array(_REGION, dtype=jnp.int32)

    info = plsc.get_sparse_core_info()
    nc, ns = info.num_cores, info.num_subcores
    nw = nc * ns
    bpw = B // nw
    wtail = W - _WMAIN
    ntf = wtail // _L
    ttail = wtail - _L

    mesh = plsc.VectorSubcoreMesh(core_axis_name="c", subcore_axis_name="s")

    @functools.partial(
        pl.kernel,
        out_type=jax.ShapeDtypeStruct((B, _C_OUT, W), x.dtype),
        mesh=mesh,
        scratch_types=[
            pltpu.VMEM((_NSLOTS, _C_OUT, _WMAIN), jnp.float32),
            pltpu.VMEM((_NSLOTS, _C_USED, wtail), jnp.float32),
            pltpu.VMEM((_NSLOTS, _C_OUT, wtail), jnp.float32),
            pltpu.VMEM((_C_OUT,), jnp.int32),
            [pltpu.SemaphoreType.DMA] * _NSLOTS,
            [pltpu.SemaphoreType.DMA] * _NSLOTS,
            [pltpu.SemaphoreType.DMA] * _NSLOTS,
            [pltpu.SemaphoreType.DMA] * _NSLOTS,
        ],
    )
    def gather_region(x_hbm, region_hbm, out_hbm, slab, tin, tout, idx,
                      g_sems, t_sems, om_sems, ot_sems):
        wid = lax.axis_index("s") * nc + lax.axis_index("c")
        b0 = wid * bpw

        pltpu.sync_copy(region_hbm, idx)

        def g_desc(i):
            s = i % _NSLOTS
            return pltpu.make_async_copy(
                x_hbm.at[b0 + i, :, pl.ds(0, _WMAIN)].at[idx],
                slab.at[s], g_sems[s])

        def t_desc(i):
            s = i % _NSLOTS
            return pltpu.make_async_copy(
                x_hbm.at[b0 + i, pl.ds(0, _C_USED), pl.ds(_WMAIN, wtail)],
                tin.at[s], t_sems[s])

        def om_desc(i):
            s = i % _NSLOTS
            return pltpu.make_async_copy(
                slab.at[s], out_hbm.at[b0 + i, :, pl.ds(0, _WMAIN)],
                om_sems[s])

        def ot_desc(i):
            s = i % _NSLOTS
            return pltpu.make_async_copy(
                tout.at[s], out_hbm.at[b0 + i, :, pl.ds(_WMAIN, wtail)],
                ot_sems[s])

        def permute_tail(s):
            def chunk(k, carry):
                off = k * _L
                for j, r in enumerate(_REGION):
                    tout[s, j, pl.ds(off, _L)] = tin[s, r, pl.ds(off, _L)]
                return carry
            lax.fori_loop(0, ntf, chunk, 0)
            for j, r in enumerate(_REGION):
                tout[s, j, pl.ds(ttail, _L)] = tin[s, r, pl.ds(ttail, _L)]

        for i in range(_NSLOTS - 1):
            g_desc(i).start()
            t_desc(i).start()
        for i in range(bpw):
            g_desc(i).wait()
            om_desc(i).start()
            t_desc(i).wait()
            permute_tail(i % _NSLOTS)
            ot_desc(i).start()
            nxt = i + _NSLOTS - 1
            if nxt < bpw:
                if nxt >= _NSLOTS:
                    om_desc(nxt - _NSLOTS).wait()
                    ot_desc(nxt - _NSLOTS).wait()
                g_desc(nxt).start()
                t_desc(nxt).start()
        for i in range(bpw - _NSLOTS, bpw):
            om_desc(i).wait()
            ot_desc(i).wait()

    out = gather_region(x3, region)
    return out.reshape(B, 1, _C_OUT, W)
